# baseline (speedup 1.0000x reference)
import jax
import jax.numpy as jnp
from jax.experimental import pallas as pl
from jax.experimental.pallas import tpu as pltpu

_LANES = 512
_BLOCK_ROWS = 512


def _focal_block_kernel(x_ref, w_ref, pos_ref, m_ref, out_ref, accn_ref, accd_ref):
    i = pl.program_id(0)

    @pl.when(i == 0)
    def _init():
        accn_ref[...] = jnp.zeros_like(accn_ref)
        accd_ref[...] = jnp.zeros_like(accd_ref)

    x = x_ref[...]
    tb = pos_ref[...].astype(jnp.float32) > 0.5
    wm = w_ref[...] * m_ref[...].astype(jnp.float32)

    z = jnp.where(tb, -x, x)
    u = jnp.exp(-jnp.abs(x))
    d = 1.0 + u
    r = 1.0 / d
    pt = jnp.where(z >= 0.0, r, 1.0 - r)
    bce = jnp.maximum(z, 0.0) + jnp.log(d)
    alpha_w = jnp.where(tb, 0.25, 0.75)
    contrib = (alpha_w * wm) * (pt * pt) * bce

    accn_ref[...] += jnp.sum(contrib.reshape(-1, 8, _LANES), axis=0)
    accd_ref[...] += jnp.sum(wm.reshape(-1, 8, _LANES), axis=0)

    @pl.when(i == pl.num_programs(0) - 1)
    def _finish():
        num = jnp.sum(accn_ref[...])
        den = jnp.sum(accd_ref[...])
        out_ref[0, 0] = num / jnp.maximum(den, 1.0)


def _as_native_2d(a):
    b, d1, d2, d3 = a.shape
    return a.transpose(0, 1, 3, 2).reshape(b * d1 * d3, d2)


def kernel(pred_occ_logit, general_cls_loss_mask_float, pos_mask, general_cls_loss_mask):
    b, _, d1, d2, d3 = pred_occ_logit.shape
    x2 = _as_native_2d(pred_occ_logit.reshape(b, d1, d2, d3))
    w2 = _as_native_2d(general_cls_loss_mask_float)
    p2 = _as_native_2d(pos_mask.view(jnp.int8))
    m2 = _as_native_2d(general_cls_loss_mask.view(jnp.int8))

    rows = x2.shape[0]
    grid = rows // _BLOCK_ROWS
    out = pl.pallas_call(
        _focal_block_kernel,
        grid=(grid,),
        in_specs=[
            pl.BlockSpec((_BLOCK_ROWS, _LANES), lambda i: (i, 0)),
            pl.BlockSpec((_BLOCK_ROWS, _LANES), lambda i: (i, 0)),
            pl.BlockSpec((_BLOCK_ROWS, _LANES), lambda i: (i, 0)),
            pl.BlockSpec((_BLOCK_ROWS, _LANES), lambda i: (i, 0)),
        ],
        out_specs=pl.BlockSpec((1, 1), lambda i: (0, 0), memory_space=pltpu.SMEM),
        out_shape=jax.ShapeDtypeStruct((1, 1), jnp.float32),
        scratch_shapes=[
            pltpu.VMEM((8, _LANES), jnp.float32),
            pltpu.VMEM((8, _LANES), jnp.float32),
        ],
    )(x2, w2, p2, m2)
    return out[0, 0]


# speedup vs baseline: 1.0917x; 1.0917x over previous
import jax
import jax.numpy as jnp
from jax.experimental import pallas as pl
from jax.experimental.pallas import tpu as pltpu

_LANES = 512
_BLOCK_ROWS = 512


def _focal_block_kernel(x_ref, w_ref, pm_ref, out_ref, accn_ref, accd_ref):
    i = pl.program_id(0)

    @pl.when(i == 0)
    def _init():
        accn_ref[...] = jnp.zeros_like(accn_ref)
        accd_ref[...] = jnp.zeros_like(accd_ref)

    x = x_ref[...]
    pm = pm_ref[...].astype(jnp.int32)
    tb = (pm & 1) > 0
    wm = w_ref[...] * (pm >> 4).astype(jnp.float32)

    z = jnp.where(tb, -x, x)
    u = jnp.exp(-jnp.abs(x))
    d = 1.0 + u
    r = 1.0 / d
    pt = jnp.where(z >= 0.0, r, 1.0 - r)
    bce = jnp.maximum(z, 0.0) + jnp.log(d)
    alpha_w = jnp.where(tb, 0.25, 0.75)
    contrib = (alpha_w * wm) * (pt * pt) * bce

    accn_ref[...] += jnp.sum(contrib.reshape(-1, 8, _LANES), axis=0)
    accd_ref[...] += jnp.sum(wm.reshape(-1, 8, _LANES), axis=0)

    @pl.when(i == pl.num_programs(0) - 1)
    def _finish():
        num = jnp.sum(accn_ref[...])
        den = jnp.sum(accd_ref[...])
        out_ref[0, 0] = num / jnp.maximum(den, 1.0)


def _as_native_2d(a):
    b, d1, d2, d3 = a.shape
    return a.transpose(0, 1, 3, 2).reshape(b * d1 * d3, d2)


def kernel(pred_occ_logit, general_cls_loss_mask_float, pos_mask, general_cls_loss_mask):
    b, _, d1, d2, d3 = pred_occ_logit.shape
    x2 = _as_native_2d(pred_occ_logit.reshape(b, d1, d2, d3))
    w2 = _as_native_2d(general_cls_loss_mask_float)
    pm2 = _as_native_2d(pos_mask.astype(jnp.int8)
                        | (general_cls_loss_mask.astype(jnp.int8) << 4))

    rows = x2.shape[0]
    grid = rows // _BLOCK_ROWS
    out = pl.pallas_call(
        _focal_block_kernel,
        grid=(grid,),
        in_specs=[
            pl.BlockSpec((_BLOCK_ROWS, _LANES), lambda i: (i, 0)),
            pl.BlockSpec((_BLOCK_ROWS, _LANES), lambda i: (i, 0)),
            pl.BlockSpec((_BLOCK_ROWS, _LANES), lambda i: (i, 0)),
        ],
        out_specs=pl.BlockSpec((1, 1), lambda i: (0, 0), memory_space=pltpu.SMEM),
        out_shape=jax.ShapeDtypeStruct((1, 1), jnp.float32),
        scratch_shapes=[
            pltpu.VMEM((8, _LANES), jnp.float32),
            pltpu.VMEM((8, _LANES), jnp.float32),
        ],
    )(x2, w2, pm2)
    return out[0, 0]


# speedup vs baseline: 1.1592x; 1.0618x over previous
import jax
import jax.numpy as jnp
from jax.experimental import pallas as pl
from jax.experimental.pallas import tpu as pltpu

_LANES = 512
_BLOCK_ROWS = 512


def _focal_block_kernel(x_ref, w_ref, pm_ref, out_ref, accn_ref, accd_ref):
    i = pl.program_id(0)

    @pl.when(i == 0)
    def _init():
        accn_ref[...] = jnp.zeros_like(accn_ref)
        accd_ref[...] = jnp.zeros_like(accd_ref)

    def body(j, carry):
        an, ad = carry
        x = x_ref[pl.ds(j * 8, 8), :]
        pm = pm_ref[pl.ds(j * 8, 8), :].astype(jnp.int32)
        tb = (pm & 1) > 0
        wm = w_ref[pl.ds(j * 8, 8), :] * (pm >> 4).astype(jnp.float32)

        z = jnp.where(tb, -x, x)
        u = jnp.exp(-jnp.abs(x))
        d = 1.0 + u
        r = 1.0 / d
        pt = jnp.where(z >= 0.0, r, 1.0 - r)
        bce = jnp.maximum(z, 0.0) + jnp.log(d)
        alpha_w = jnp.where(tb, 0.25, 0.75)
        contrib = (alpha_w * wm) * (pt * pt) * bce
        return an + contrib, ad + wm

    zero = jnp.zeros((8, _LANES), jnp.float32)
    an, ad = jax.lax.fori_loop(0, _BLOCK_ROWS // 8, body, (zero, zero),
                               unroll=2)
    accn_ref[...] += an
    accd_ref[...] += ad

    @pl.when(i == pl.num_programs(0) - 1)
    def _finish():
        num = jnp.sum(accn_ref[...])
        den = jnp.sum(accd_ref[...])
        out_ref[0, 0] = num / jnp.maximum(den, 1.0)


def _as_native_2d(a):
    b, d1, d2, d3 = a.shape
    return a.transpose(0, 1, 3, 2).reshape(b * d1 * d3, d2)


def kernel(pred_occ_logit, general_cls_loss_mask_float, pos_mask, general_cls_loss_mask):
    b, _, d1, d2, d3 = pred_occ_logit.shape
    x2 = _as_native_2d(pred_occ_logit.reshape(b, d1, d2, d3))
    w2 = _as_native_2d(general_cls_loss_mask_float)
    pm2 = _as_native_2d(pos_mask.astype(jnp.int8)
                        | (general_cls_loss_mask.astype(jnp.int8) << 4))

    rows = x2.shape[0]
    grid = rows // _BLOCK_ROWS
    out = pl.pallas_call(
        _focal_block_kernel,
        grid=(grid,),
        in_specs=[
            pl.BlockSpec((_BLOCK_ROWS, _LANES), lambda i: (i, 0)),
            pl.BlockSpec((_BLOCK_ROWS, _LANES), lambda i: (i, 0)),
            pl.BlockSpec((_BLOCK_ROWS, _LANES), lambda i: (i, 0)),
        ],
        out_specs=pl.BlockSpec((1, 1), lambda i: (0, 0), memory_space=pltpu.SMEM),
        out_shape=jax.ShapeDtypeStruct((1, 1), jnp.float32),
        scratch_shapes=[
            pltpu.VMEM((8, _LANES), jnp.float32),
            pltpu.VMEM((8, _LANES), jnp.float32),
        ],
    )(x2, w2, pm2)
    return out[0, 0]


# speedup vs baseline: 1.3841x; 1.1940x over previous
import jax
import jax.numpy as jnp
from jax.experimental import pallas as pl
from jax.experimental.pallas import tpu as pltpu

_LANES = 512
_BLOCK_ROWS = 1024


def _focal_block_kernel(x_ref, w_ref, pm_ref, out_ref, accn_ref, accd_ref):
    i = pl.program_id(0)

    @pl.when(i == 0)
    def _init():
        accn_ref[...] = jnp.zeros_like(accn_ref)
        accd_ref[...] = jnp.zeros_like(accd_ref)

    def body(j, carry):
        an, ad = carry
        x = x_ref[pl.ds(j * 8, 8), :]
        pm = pm_ref[pl.ds(j * 8, 8), :].astype(jnp.int32)
        tb = (pm & 1) > 0
        wm = w_ref[pl.ds(j * 8, 8), :] * (pm >> 4).astype(jnp.float32)

        z = jnp.where(tb, -x, x)
        u = jnp.exp(-jnp.abs(x))
        d = 1.0 + u
        r = 1.0 / d
        pt = jnp.where(z >= 0.0, r, 1.0 - r)
        bce = jnp.maximum(z, 0.0) + jnp.log(d)
        alpha_w = jnp.where(tb, 0.25, 0.75)
        contrib = (alpha_w * wm) * (pt * pt) * bce
        return an + contrib, ad + wm

    zero = jnp.zeros((8, _LANES), jnp.float32)
    an, ad = jax.lax.fori_loop(0, _BLOCK_ROWS // 8, body, (zero, zero),
                               unroll=4)
    accn_ref[...] += an
    accd_ref[...] += ad

    @pl.when(i == pl.num_programs(0) - 1)
    def _finish():
        num = jnp.sum(accn_ref[...])
        den = jnp.sum(accd_ref[...])
        out_ref[0, 0] = num / jnp.maximum(den, 1.0)


def _as_native_2d(a):
    b, d1, d2, d3 = a.shape
    return a.transpose(0, 1, 3, 2).reshape(b * d1 * d3, d2)


def kernel(pred_occ_logit, general_cls_loss_mask_float, pos_mask, general_cls_loss_mask):
    b, _, d1, d2, d3 = pred_occ_logit.shape
    x2 = _as_native_2d(pred_occ_logit.reshape(b, d1, d2, d3))
    w2 = _as_native_2d(general_cls_loss_mask_float)
    pm2 = _as_native_2d(pos_mask.astype(jnp.int8)
                        | (general_cls_loss_mask.astype(jnp.int8) << 4))

    rows = x2.shape[0]
    grid = rows // _BLOCK_ROWS
    out = pl.pallas_call(
        _focal_block_kernel,
        grid=(grid,),
        in_specs=[
            pl.BlockSpec((_BLOCK_ROWS, _LANES), lambda i: (i, 0)),
            pl.BlockSpec((_BLOCK_ROWS, _LANES), lambda i: (i, 0)),
            pl.BlockSpec((_BLOCK_ROWS, _LANES), lambda i: (i, 0)),
        ],
        out_specs=pl.BlockSpec((1, 1), lambda i: (0, 0), memory_space=pltpu.SMEM),
        out_shape=jax.ShapeDtypeStruct((1, 1), jnp.float32),
        scratch_shapes=[
            pltpu.VMEM((8, _LANES), jnp.float32),
            pltpu.VMEM((8, _LANES), jnp.float32),
        ],
    )(x2, w2, pm2)
    return out[0, 0]


# speedup vs baseline: 1.4387x; 1.0394x over previous
import jax
import jax.numpy as jnp
from jax.experimental import pallas as pl
from jax.experimental.pallas import tpu as pltpu

_LANES = 512
_BLOCK_ROWS = 2048


def _focal_block_kernel(x_ref, w_ref, pm_ref, out_ref, accn_ref, accd_ref):
    i = pl.program_id(0)

    @pl.when(i == 0)
    def _init():
        accn_ref[...] = jnp.zeros_like(accn_ref)
        accd_ref[...] = jnp.zeros_like(accd_ref)

    def body(j, carry):
        an, ad = carry
        x = x_ref[pl.ds(j * 8, 8), :]
        pm = pm_ref[pl.ds(j * 8, 8), :].astype(jnp.int32)
        tb = (pm & 1) > 0
        wm = w_ref[pl.ds(j * 8, 8), :] * (pm >> 4).astype(jnp.float32)

        z = jnp.where(tb, -x, x)
        u = jnp.exp(-jnp.abs(x))
        d = 1.0 + u
        r = 1.0 / d
        pt = jnp.where(z >= 0.0, r, 1.0 - r)
        bce = jnp.maximum(z, 0.0) + jnp.log(d)
        alpha_w = jnp.where(tb, 0.25, 0.75)
        contrib = (alpha_w * wm) * (pt * pt) * bce
        return an + contrib, ad + wm

    zero = jnp.zeros((8, _LANES), jnp.float32)
    an, ad = jax.lax.fori_loop(0, _BLOCK_ROWS // 8, body, (zero, zero),
                               unroll=4)
    accn_ref[...] += an
    accd_ref[...] += ad

    @pl.when(i == pl.num_programs(0) - 1)
    def _finish():
        num = jnp.sum(accn_ref[...])
        den = jnp.sum(accd_ref[...])
        out_ref[0, 0] = num / jnp.maximum(den, 1.0)


def _as_native_2d(a):
    b, d1, d2, d3 = a.shape
    return a.transpose(0, 1, 3, 2).reshape(b * d1 * d3, d2)


def kernel(pred_occ_logit, general_cls_loss_mask_float, pos_mask, general_cls_loss_mask):
    b, _, d1, d2, d3 = pred_occ_logit.shape
    x2 = _as_native_2d(pred_occ_logit.reshape(b, d1, d2, d3))
    w2 = _as_native_2d(general_cls_loss_mask_float)
    pm2 = _as_native_2d(pos_mask.astype(jnp.int8)
                        | (general_cls_loss_mask.astype(jnp.int8) << 4))

    rows = x2.shape[0]
    grid = rows // _BLOCK_ROWS
    out = pl.pallas_call(
        _focal_block_kernel,
        grid=(grid,),
        in_specs=[
            pl.BlockSpec((_BLOCK_ROWS, _LANES), lambda i: (i, 0)),
            pl.BlockSpec((_BLOCK_ROWS, _LANES), lambda i: (i, 0)),
            pl.BlockSpec((_BLOCK_ROWS, _LANES), lambda i: (i, 0)),
        ],
        out_specs=pl.BlockSpec((1, 1), lambda i: (0, 0), memory_space=pltpu.SMEM),
        out_shape=jax.ShapeDtypeStruct((1, 1), jnp.float32),
        scratch_shapes=[
            pltpu.VMEM((8, _LANES), jnp.float32),
            pltpu.VMEM((8, _LANES), jnp.float32),
        ],
    )(x2, w2, pm2)
    return out[0, 0]


# speedup vs baseline: 1.4842x; 1.0316x over previous
import jax
import jax.numpy as jnp
from jax.experimental import pallas as pl
from jax.experimental.pallas import tpu as pltpu

_LANES = 512
_BLOCK_ROWS = 2048


def _focal_block_kernel(x_ref, w_ref, pm_ref, out_ref, accn_ref, accd_ref):
    i = pl.program_id(0)

    @pl.when(i == 0)
    def _init():
        accn_ref[...] = jnp.zeros_like(accn_ref)
        accd_ref[...] = jnp.zeros_like(accd_ref)

    def body(j, carry):
        an, ad = carry
        x = x_ref[pl.ds(j * 8, 8), :]
        pm = pm_ref[pl.ds(j * 8, 8), :].astype(jnp.int32)
        tb = (pm & 1) > 0
        wm = w_ref[pl.ds(j * 8, 8), :] * (pm >> 4).astype(jnp.float32)

        z = jnp.where(tb, -x, x)
        u = jnp.exp(-jnp.abs(x))
        d = 1.0 + u
        r = 1.0 / d
        pt = jnp.where(z >= 0.0, r, 1.0 - r)
        bce = jnp.maximum(z, 0.0) + jnp.log(d)
        alpha_w = jnp.where(tb, 0.25, 0.75)
        contrib = (alpha_w * wm) * (pt * pt) * bce
        return an + contrib, ad + wm

    zero = jnp.zeros((8, _LANES), jnp.float32)
    an, ad = jax.lax.fori_loop(0, _BLOCK_ROWS // 8, body, (zero, zero),
                               unroll=8)
    accn_ref[...] += an
    accd_ref[...] += ad

    @pl.when(i == pl.num_programs(0) - 1)
    def _finish():
        num = jnp.sum(accn_ref[...])
        den = jnp.sum(accd_ref[...])
        out_ref[0, 0] = num / jnp.maximum(den, 1.0)


def _as_native_2d(a):
    b, d1, d2, d3 = a.shape
    return a.transpose(0, 1, 3, 2).reshape(b * d1 * d3, d2)


def kernel(pred_occ_logit, general_cls_loss_mask_float, pos_mask, general_cls_loss_mask):
    b, _, d1, d2, d3 = pred_occ_logit.shape
    x2 = _as_native_2d(pred_occ_logit.reshape(b, d1, d2, d3))
    w2 = _as_native_2d(general_cls_loss_mask_float)
    pm2 = _as_native_2d(pos_mask.astype(jnp.int8)
                        | (general_cls_loss_mask.astype(jnp.int8) << 4))

    rows = x2.shape[0]
    grid = rows // _BLOCK_ROWS
    out = pl.pallas_call(
        _focal_block_kernel,
        grid=(grid,),
        in_specs=[
            pl.BlockSpec((_BLOCK_ROWS, _LANES), lambda i: (i, 0)),
            pl.BlockSpec((_BLOCK_ROWS, _LANES), lambda i: (i, 0)),
            pl.BlockSpec((_BLOCK_ROWS, _LANES), lambda i: (i, 0)),
        ],
        out_specs=pl.BlockSpec((1, 1), lambda i: (0, 0), memory_space=pltpu.SMEM),
        out_shape=jax.ShapeDtypeStruct((1, 1), jnp.float32),
        scratch_shapes=[
            pltpu.VMEM((8, _LANES), jnp.float32),
            pltpu.VMEM((8, _LANES), jnp.float32),
        ],
    )(x2, w2, pm2)
    return out[0, 0]
